# initial kernel scaffold (unmeasured)
import jax
import jax.numpy as jnp
from jax import lax
from jax.experimental import pallas as pl
from jax.experimental.pallas import tpu as pltpu

N_DEV = 4
M_PER = 1024
N_COLS = 8192
W = 2048
S = N_COLS // W
N_HOPS = N_DEV - 1


def kernel(x, w_mat):
    partial = jnp.dot(x, w_mat, preferred_element_type=jnp.float32)

    def body(partial_ref, out_ref, send_buf, recv_buf, local_buf,
             send_sem, recv_sem, local_sem, credit_sem):
        q = lax.axis_index("i")
        left = (q + N_DEV - 1) % N_DEV
        right = (q + 1) % N_DEV

        barrier = pltpu.get_barrier_semaphore()
        for nbr in (left, right):
            pl.semaphore_signal(barrier, inc=1, device_id=(nbr,),
                                device_id_type=pl.DeviceIdType.MESH)
        pl.semaphore_wait(barrier, 2)

        pl.semaphore_signal(credit_sem, inc=1, device_id=(left,),
                            device_id_type=pl.DeviceIdType.MESH)

        for j in range(S):
            col = j * W
            init = pltpu.make_async_copy(
                partial_ref.at[pl.ds(left * M_PER, M_PER), pl.ds(col, W)],
                send_buf, local_sem)
            init.start()
            init.wait()
            for h in range(N_HOPS):
                c = (q + 2 - h) % N_DEV
                fetch = pltpu.make_async_copy(
                    partial_ref.at[pl.ds(c * M_PER, M_PER), pl.ds(col, W)],
                    local_buf, local_sem)
                fetch.start()
                pl.semaphore_wait(credit_sem, 1)
                rdma = pltpu.make_async_remote_copy(
                    src_ref=send_buf, dst_ref=recv_buf,
                    send_sem=send_sem, recv_sem=recv_sem,
                    device_id=(right,), device_id_type=pl.DeviceIdType.MESH)
                rdma.start()
                rdma.wait()
                fetch.wait()
                if h < N_HOPS - 1:
                    send_buf[...] = recv_buf[...] + local_buf[...]
                else:
                    out_ref[:, pl.ds(col, W)] = jnp.maximum(
                        recv_buf[...] + local_buf[...], 0.0)
                if not (j == S - 1 and h == N_HOPS - 1):
                    pl.semaphore_signal(credit_sem, inc=1, device_id=(left,),
                                        device_id_type=pl.DeviceIdType.MESH)

    return pl.pallas_call(
        body,
        out_shape=jax.ShapeDtypeStruct((M_PER, N_COLS), jnp.float32),
        in_specs=[pl.BlockSpec(memory_space=pltpu.ANY)],
        out_specs=pl.BlockSpec(memory_space=pltpu.VMEM),
        scratch_shapes=[
            pltpu.VMEM((M_PER, W), jnp.float32),
            pltpu.VMEM((M_PER, W), jnp.float32),
            pltpu.VMEM((M_PER, W), jnp.float32),
            pltpu.SemaphoreType.DMA,
            pltpu.SemaphoreType.DMA,
            pltpu.SemaphoreType.DMA,
            pltpu.SemaphoreType.REGULAR,
        ],
        compiler_params=pltpu.CompilerParams(collective_id=0),
    )(partial)


# baseline (device time: 1249733 ns/iter reference)
import jax
import jax.numpy as jnp
from jax import lax
from jax.experimental import pallas as pl
from jax.experimental.pallas import tpu as pltpu

N_DEV = 4
M_PER = 1024
N_COLS = 8192
W = 2048
S = N_COLS // W
N_HOPS = N_DEV - 1


def kernel(x, w_mat):
    partial = jnp.dot(x, w_mat, preferred_element_type=jnp.float32)

    def body(partial_ref, out_ref, send_buf, recv_buf, local_buf,
             send_sem, recv_sem, local_sem, out_sem, credit_sem):
        q = lax.axis_index("i")
        left = (q + N_DEV - 1) % N_DEV
        right = (q + 1) % N_DEV

        barrier = pltpu.get_barrier_semaphore()
        for nbr in (left, right):
            pl.semaphore_signal(barrier, inc=1, device_id=(nbr,),
                                device_id_type=pl.DeviceIdType.MESH)
        pl.semaphore_wait(barrier, 2)

        pl.semaphore_signal(credit_sem, inc=1, device_id=(left,),
                            device_id_type=pl.DeviceIdType.MESH)

        for j in range(S):
            col = j * W
            init = pltpu.make_async_copy(
                partial_ref.at[pl.ds(left * M_PER, M_PER), pl.ds(col, W)],
                send_buf, local_sem)
            init.start()
            init.wait()
            for h in range(N_HOPS):
                c = (q + 2 - h) % N_DEV
                fetch = pltpu.make_async_copy(
                    partial_ref.at[pl.ds(c * M_PER, M_PER), pl.ds(col, W)],
                    local_buf, local_sem)
                fetch.start()
                pl.semaphore_wait(credit_sem, 1)
                rdma = pltpu.make_async_remote_copy(
                    src_ref=send_buf, dst_ref=recv_buf,
                    send_sem=send_sem, recv_sem=recv_sem,
                    device_id=(right,), device_id_type=pl.DeviceIdType.MESH)
                rdma.start()
                rdma.wait()
                fetch.wait()
                if h < N_HOPS - 1:
                    send_buf[...] = recv_buf[...] + local_buf[...]
                else:
                    send_buf[...] = jnp.maximum(
                        recv_buf[...] + local_buf[...], 0.0)
                    out_copy = pltpu.make_async_copy(
                        send_buf, out_ref.at[:, pl.ds(col, W)], out_sem)
                    out_copy.start()
                    out_copy.wait()
                if not (j == S - 1 and h == N_HOPS - 1):
                    pl.semaphore_signal(credit_sem, inc=1, device_id=(left,),
                                        device_id_type=pl.DeviceIdType.MESH)

    return pl.pallas_call(
        body,
        out_shape=jax.ShapeDtypeStruct((M_PER, N_COLS), jnp.float32),
        in_specs=[pl.BlockSpec(memory_space=pl.ANY)],
        out_specs=pl.BlockSpec(memory_space=pl.ANY),
        scratch_shapes=[
            pltpu.VMEM((M_PER, W), jnp.float32),
            pltpu.VMEM((M_PER, W), jnp.float32),
            pltpu.VMEM((M_PER, W), jnp.float32),
            pltpu.SemaphoreType.DMA,
            pltpu.SemaphoreType.DMA,
            pltpu.SemaphoreType.DMA,
            pltpu.SemaphoreType.DMA,
            pltpu.SemaphoreType.REGULAR,
        ],
        compiler_params=pltpu.CompilerParams(collective_id=0),
    )(partial)


# device time: 713168 ns/iter; 1.7524x vs baseline; 1.7524x over previous
import jax
import jax.numpy as jnp
from jax import lax
from jax.experimental import pallas as pl
from jax.experimental.pallas import tpu as pltpu

N_DEV = 4
M_PER = 1024
N_COLS = 8192
W = 2048
H = W // 2
S = N_COLS // W
N_HOPS = N_DEV - 1


def kernel(x, w_mat):
    partial = jnp.dot(x, w_mat, preferred_element_type=jnp.float32)

    def body(partial_ref, out_ref,
             send_r, recv_r, send_l, recv_l, local_r, local_l,
             send_sem_r, recv_sem_r, send_sem_l, recv_sem_l,
             fetch_sem_r, fetch_sem_l, out_sem,
             credit_r, credit_l):
        q = lax.axis_index("i")
        left = (q + N_DEV - 1) % N_DEV
        right = (q + 1) % N_DEV

        barrier = pltpu.get_barrier_semaphore()
        for nbr in (left, right):
            pl.semaphore_signal(barrier, inc=1, device_id=(nbr,),
                                device_id_type=pl.DeviceIdType.MESH)
        pl.semaphore_wait(barrier, 2)

        pl.semaphore_signal(credit_r, inc=1, device_id=(left,),
                            device_id_type=pl.DeviceIdType.MESH)
        pl.semaphore_signal(credit_l, inc=1, device_id=(right,),
                            device_id_type=pl.DeviceIdType.MESH)

        def fetch(chunk, col, width, dst, sem):
            cp = pltpu.make_async_copy(
                partial_ref.at[pl.ds(chunk * M_PER, M_PER),
                               pl.ds(col, width)],
                dst, sem)
            cp.start()
            return cp

        for j in range(S):
            col_l = j * W
            col_r = j * W + H
            init_r = fetch((q + 3) % N_DEV, col_r, H, send_r, fetch_sem_r)
            init_l = fetch((q + 1) % N_DEV, col_l, H, send_l, fetch_sem_l)
            init_r.wait()
            init_l.wait()
            for h in range(N_HOPS):
                cr = (q + 2 - h) % N_DEV
                cl = (q + 2 + h) % N_DEV
                f_r = fetch(cr, col_r, H, local_r, fetch_sem_r)
                f_l = fetch(cl, col_l, H, local_l, fetch_sem_l)
                pl.semaphore_wait(credit_r, 1)
                rdma_r = pltpu.make_async_remote_copy(
                    src_ref=send_r, dst_ref=recv_r,
                    send_sem=send_sem_r, recv_sem=recv_sem_r,
                    device_id=(right,), device_id_type=pl.DeviceIdType.MESH)
                rdma_r.start()
                pl.semaphore_wait(credit_l, 1)
                rdma_l = pltpu.make_async_remote_copy(
                    src_ref=send_l, dst_ref=recv_l,
                    send_sem=send_sem_l, recv_sem=recv_sem_l,
                    device_id=(left,), device_id_type=pl.DeviceIdType.MESH)
                rdma_l.start()
                rdma_r.wait()
                rdma_l.wait()
                f_r.wait()
                f_l.wait()
                if h < N_HOPS - 1:
                    send_r[...] = recv_r[...] + local_r[...]
                    send_l[...] = recv_l[...] + local_l[...]
                else:
                    send_r[...] = jnp.maximum(recv_r[...] + local_r[...], 0.0)
                    send_l[...] = jnp.maximum(recv_l[...] + local_l[...], 0.0)
                    out_r = pltpu.make_async_copy(
                        send_r, out_ref.at[:, pl.ds(col_r, H)], out_sem)
                    out_r.start()
                    out_r.wait()
                    out_l = pltpu.make_async_copy(
                        send_l, out_ref.at[:, pl.ds(col_l, H)], out_sem)
                    out_l.start()
                    out_l.wait()
                if not (j == S - 1 and h == N_HOPS - 1):
                    pl.semaphore_signal(credit_r, inc=1, device_id=(left,),
                                        device_id_type=pl.DeviceIdType.MESH)
                    pl.semaphore_signal(credit_l, inc=1, device_id=(right,),
                                        device_id_type=pl.DeviceIdType.MESH)

    return pl.pallas_call(
        body,
        out_shape=jax.ShapeDtypeStruct((M_PER, N_COLS), jnp.float32),
        in_specs=[pl.BlockSpec(memory_space=pl.ANY)],
        out_specs=pl.BlockSpec(memory_space=pl.ANY),
        scratch_shapes=[
            pltpu.VMEM((M_PER, H), jnp.float32),
            pltpu.VMEM((M_PER, H), jnp.float32),
            pltpu.VMEM((M_PER, H), jnp.float32),
            pltpu.VMEM((M_PER, H), jnp.float32),
            pltpu.VMEM((M_PER, H), jnp.float32),
            pltpu.VMEM((M_PER, H), jnp.float32),
            pltpu.SemaphoreType.DMA,
            pltpu.SemaphoreType.DMA,
            pltpu.SemaphoreType.DMA,
            pltpu.SemaphoreType.DMA,
            pltpu.SemaphoreType.DMA,
            pltpu.SemaphoreType.DMA,
            pltpu.SemaphoreType.DMA,
            pltpu.SemaphoreType.REGULAR,
            pltpu.SemaphoreType.REGULAR,
        ],
        compiler_params=pltpu.CompilerParams(collective_id=0),
    )(partial)


# device time: 662292 ns/iter; 1.8870x vs baseline; 1.0768x over previous
import jax
import jax.numpy as jnp
from jax import lax
from jax.experimental import pallas as pl
from jax.experimental.pallas import tpu as pltpu

N_DEV = 4
M_PER = 1024
K = 1024
N_COLS = 8192
W = 1024
H = W // 2
S = N_COLS // W
N_HOPS = N_DEV - 1
N_GEMMS = 4 * S
CHUNK_ORDER = (3, 1, 2, 0)
_MESH = pl.DeviceIdType.MESH


def kernel(x, w_mat):
    def body(x_hbm, w_hbm, out_ref,
             xst, wst, pbuf, send_r, recv_r, send_l, recv_l,
             xsem, wsem,
             send_sem_r, recv_sem_r, send_sem_l, recv_sem_l,
             out_sem_r, out_sem_l, credit_r, credit_l):
        q = lax.axis_index("i")
        left = (q + N_DEV - 1) % N_DEV
        right = (q + 1) % N_DEV

        barrier = pltpu.get_barrier_semaphore()
        for nbr in (left, right):
            pl.semaphore_signal(barrier, inc=1, device_id=(nbr,),
                                device_id_type=_MESH)
        pl.semaphore_wait(barrier, 2)
        pl.semaphore_signal(credit_r, inc=1, device_id=(left,),
                            device_id_type=_MESH)
        pl.semaphore_signal(credit_l, inc=1, device_id=(right,),
                            device_id_type=_MESH)

        pending_x = {}
        pending_w = {}

        def chunk_of(g):
            return (q + CHUNK_ORDER[g % 4]) % N_DEV

        def start_xfetch(g):
            cp = pltpu.make_async_copy(
                x_hbm.at[pl.ds(chunk_of(g) * M_PER, M_PER), :],
                xst.at[g % 2], xsem.at[g % 2])
            cp.start()
            pending_x[g] = cp

        def start_wfetch(j):
            cp = pltpu.make_async_copy(
                w_hbm.at[:, pl.ds(j * W, W)], wst.at[j % 2], wsem.at[j % 2])
            cp.start()
            pending_w[j] = cp

        def emit_gemm(g):
            j = g // 4
            if g + 1 < N_GEMMS:
                start_xfetch(g + 1)
            if j in pending_w:
                pending_w.pop(j).wait()
            pending_x.pop(g).wait()
            pbuf[j % 2, pl.ds(chunk_of(g) * M_PER, M_PER), :] = jnp.dot(
                xst[g % 2, :, :], wst[j % 2, :, :],
                preferred_element_type=jnp.float32)

        start_wfetch(0)
        start_xfetch(0)
        for g in range(4):
            emit_gemm(g)

        for j in range(S):
            slot = j % 2
            if j + 1 < S:
                start_wfetch(j + 1)
            for h in range(N_HOPS):
                if h == 0:
                    src_r = pbuf.at[slot,
                                    pl.ds(((q + 3) % N_DEV) * M_PER, M_PER),
                                    pl.ds(H, H)]
                    src_l = pbuf.at[slot,
                                    pl.ds(((q + 1) % N_DEV) * M_PER, M_PER),
                                    pl.ds(0, H)]
                else:
                    src_r, src_l = send_r, send_l
                pl.semaphore_wait(credit_r, 1)
                rdma_r = pltpu.make_async_remote_copy(
                    src_ref=src_r, dst_ref=recv_r,
                    send_sem=send_sem_r, recv_sem=recv_sem_r,
                    device_id=(right,), device_id_type=_MESH)
                rdma_r.start()
                pl.semaphore_wait(credit_l, 1)
                rdma_l = pltpu.make_async_remote_copy(
                    src_ref=src_l, dst_ref=recv_l,
                    send_sem=send_sem_l, recv_sem=recv_sem_l,
                    device_id=(left,), device_id_type=_MESH)
                rdma_l.start()
                if j + 1 < S:
                    for g_idx in ((0, 1), (2,), (3,))[h]:
                        emit_gemm(4 * (j + 1) + g_idx)
                rdma_r.wait()
                rdma_l.wait()
                cr = (q + 2 - h) % N_DEV
                cl = (q + 2 + h) % N_DEV
                loc_r = pbuf[slot, pl.ds(cr * M_PER, M_PER), pl.ds(H, H)]
                loc_l = pbuf[slot, pl.ds(cl * M_PER, M_PER), pl.ds(0, H)]
                if h < N_HOPS - 1:
                    send_r[...] = recv_r[...] + loc_r
                    send_l[...] = recv_l[...] + loc_l
                else:
                    send_r[...] = jnp.maximum(recv_r[...] + loc_r, 0.0)
                    send_l[...] = jnp.maximum(recv_l[...] + loc_l, 0.0)
                    out_r = pltpu.make_async_copy(
                        send_r, out_ref.at[:, pl.ds(j * W + H, H)], out_sem_r)
                    out_r.start()
                    out_l = pltpu.make_async_copy(
                        send_l, out_ref.at[:, pl.ds(j * W, H)], out_sem_l)
                    out_l.start()
                    out_r.wait()
                    out_l.wait()
                if not (j == S - 1 and h == N_HOPS - 1):
                    pl.semaphore_signal(credit_r, inc=1, device_id=(left,),
                                        device_id_type=_MESH)
                    pl.semaphore_signal(credit_l, inc=1, device_id=(right,),
                                        device_id_type=_MESH)

    return pl.pallas_call(
        body,
        out_shape=jax.ShapeDtypeStruct((M_PER, N_COLS), jnp.float32),
        in_specs=[pl.BlockSpec(memory_space=pl.ANY),
                  pl.BlockSpec(memory_space=pl.ANY)],
        out_specs=pl.BlockSpec(memory_space=pl.ANY),
        scratch_shapes=[
            pltpu.VMEM((2, M_PER, K), jnp.float32),
            pltpu.VMEM((2, K, W), jnp.float32),
            pltpu.VMEM((2, N_DEV * M_PER, W), jnp.float32),
            pltpu.VMEM((M_PER, H), jnp.float32),
            pltpu.VMEM((M_PER, H), jnp.float32),
            pltpu.VMEM((M_PER, H), jnp.float32),
            pltpu.VMEM((M_PER, H), jnp.float32),
            pltpu.SemaphoreType.DMA((2,)),
            pltpu.SemaphoreType.DMA((2,)),
            pltpu.SemaphoreType.DMA,
            pltpu.SemaphoreType.DMA,
            pltpu.SemaphoreType.DMA,
            pltpu.SemaphoreType.DMA,
            pltpu.SemaphoreType.DMA,
            pltpu.SemaphoreType.DMA,
            pltpu.SemaphoreType.REGULAR,
            pltpu.SemaphoreType.REGULAR,
        ],
        compiler_params=pltpu.CompilerParams(
            collective_id=0,
            vmem_limit_bytes=100 * 1024 * 1024,
        ),
    )(x, w_mat)


# device time: 578447 ns/iter; 2.1605x vs baseline; 1.1449x over previous
import jax
import jax.numpy as jnp
from jax import lax
from jax.experimental import pallas as pl
from jax.experimental.pallas import tpu as pltpu

N_DEV = 4
M_PER = 1024
K = 1024
N_COLS = 8192
W = 1024
H = W // 2
S = N_COLS // W
NS = 2
RS = M_PER // NS
N_GEMMS = 4 * S
CHUNK_ORDER = (3, 1, 2, 0)
_MESH = pl.DeviceIdType.MESH


def kernel(x, w_mat):
    def body(x_hbm, w_hbm, out_ref,
             xst, wst, pbuf, send_r, recv_r, send_l, recv_l,
             xsem, wsem,
             send_sem_r, recv_sem_r, send_sem_l, recv_sem_l,
             out_sem_r, out_sem_l, credit_r, credit_l):
        q = lax.axis_index("i")
        left = (q + N_DEV - 1) % N_DEV
        right = (q + 1) % N_DEV

        barrier = pltpu.get_barrier_semaphore()
        for nbr in (left, right):
            pl.semaphore_signal(barrier, inc=1, device_id=(nbr,),
                                device_id_type=_MESH)
        pl.semaphore_wait(barrier, 2)
        pl.semaphore_signal(credit_r, inc=NS, device_id=(left,),
                            device_id_type=_MESH)
        pl.semaphore_signal(credit_l, inc=NS, device_id=(right,),
                            device_id_type=_MESH)

        pending_x = {}
        pending_w = {}
        pending_out = {}
        rd_r = {}
        rd_l = {}

        def chunk_of(g):
            return (q + CHUNK_ORDER[g % 4]) % N_DEV

        def start_xfetch(g):
            cp = pltpu.make_async_copy(
                x_hbm.at[pl.ds(chunk_of(g) * M_PER, M_PER), :],
                xst.at[g % 2], xsem.at[g % 2])
            cp.start()
            pending_x[g] = cp

        def start_wfetch(j):
            cp = pltpu.make_async_copy(
                w_hbm.at[:, pl.ds(j * W, W)], wst.at[j % 2], wsem.at[j % 2])
            cp.start()
            pending_w[j] = cp

        def emit_gemm(g):
            j = g // 4
            if g + 1 < N_GEMMS:
                start_xfetch(g + 1)
            if j in pending_w:
                pending_w.pop(j).wait()
            pending_x.pop(g).wait()
            pbuf[j % 2, pl.ds(chunk_of(g) * M_PER, M_PER), :] = jnp.dot(
                xst[g % 2, :, :], wst[j % 2, :, :],
                preferred_element_type=jnp.float32)

        def start_fwd(j, h, s):
            slot = j % 2
            if h == 0:
                src_r = pbuf.at[slot,
                                pl.ds(((q + 3) % N_DEV) * M_PER + s * RS, RS),
                                pl.ds(H, H)]
                src_l = pbuf.at[slot,
                                pl.ds(((q + 1) % N_DEV) * M_PER + s * RS, RS),
                                pl.ds(0, H)]
            else:
                src_r = send_r.at[pl.ds(s * RS, RS), :]
                src_l = send_l.at[pl.ds(s * RS, RS), :]
            pl.semaphore_wait(credit_r, 1)
            d = pltpu.make_async_remote_copy(
                src_ref=src_r, dst_ref=recv_r.at[pl.ds(s * RS, RS), :],
                send_sem=send_sem_r.at[s], recv_sem=recv_sem_r.at[s],
                device_id=(right,), device_id_type=_MESH)
            d.start()
            rd_r[(j, h, s)] = d
            pl.semaphore_wait(credit_l, 1)
            d = pltpu.make_async_remote_copy(
                src_ref=src_l, dst_ref=recv_l.at[pl.ds(s * RS, RS), :],
                send_sem=send_sem_l.at[s], recv_sem=recv_sem_l.at[s],
                device_id=(left,), device_id_type=_MESH)
            d.start()
            rd_l[(j, h, s)] = d

        def consume(j, h, s):
            slot = j % 2
            dr = rd_r.pop((j, h, s))
            dl = rd_l.pop((j, h, s))
            dr.wait_recv()
            dr.wait_send()
            dl.wait_recv()
            dl.wait_send()
            if h == 0 and s == 0:
                for d in pending_out.values():
                    d.wait()
                pending_out.clear()
            rows = pl.ds(s * RS, RS)
            cr = (q + 2 - h) % N_DEV
            cl = (q + 2 + h) % N_DEV
            loc_r = pbuf[slot, pl.ds(cr * M_PER + s * RS, RS), pl.ds(H, H)]
            loc_l = pbuf[slot, pl.ds(cl * M_PER + s * RS, RS), pl.ds(0, H)]
            if h < N_HOPS_LAST:
                send_r[rows, :] = recv_r[rows, :] + loc_r
                send_l[rows, :] = recv_l[rows, :] + loc_l
            else:
                send_r[rows, :] = jnp.maximum(recv_r[rows, :] + loc_r, 0.0)
                send_l[rows, :] = jnp.maximum(recv_l[rows, :] + loc_l, 0.0)
            if not (j == S - 1 and h == N_HOPS_LAST):
                pl.semaphore_signal(credit_r, inc=1, device_id=(left,),
                                    device_id_type=_MESH)
                pl.semaphore_signal(credit_l, inc=1, device_id=(right,),
                                    device_id_type=_MESH)
            nj, nh = (j, h + 1) if h < N_HOPS_LAST else (j + 1, 0)
            if nj < S:
                start_fwd(nj, nh, s)
            if j + 1 < S:
                base = 4 * (j + 1)
                for off in {(0, 0): (0,), (0, 1): (1,),
                            (1, 0): (2,), (2, 0): (3,)}.get((h, s), ()):
                    emit_gemm(base + off)
            if h == N_HOPS_LAST and s == NS - 1:
                d = pltpu.make_async_copy(
                    send_r, out_ref.at[:, pl.ds(j * W + H, H)], out_sem_r)
                d.start()
                pending_out['r'] = d
                d = pltpu.make_async_copy(
                    send_l, out_ref.at[:, pl.ds(j * W, H)], out_sem_l)
                d.start()
                pending_out['l'] = d

        N_HOPS_LAST = N_DEV - 2
        start_wfetch(0)
        start_xfetch(0)
        emit_gemm(0)
        emit_gemm(1)
        for s in range(NS):
            start_fwd(0, 0, s)
        emit_gemm(2)
        emit_gemm(3)
        for j in range(S):
            if j + 1 < S:
                start_wfetch(j + 1)
            for h in range(N_DEV - 1):
                for s in range(NS):
                    consume(j, h, s)
        for d in pending_out.values():
            d.wait()

    return pl.pallas_call(
        body,
        out_shape=jax.ShapeDtypeStruct((M_PER, N_COLS), jnp.float32),
        in_specs=[pl.BlockSpec(memory_space=pl.ANY),
                  pl.BlockSpec(memory_space=pl.ANY)],
        out_specs=pl.BlockSpec(memory_space=pl.ANY),
        scratch_shapes=[
            pltpu.VMEM((2, M_PER, K), jnp.float32),
            pltpu.VMEM((2, K, W), jnp.float32),
            pltpu.VMEM((2, N_DEV * M_PER, W), jnp.float32),
            pltpu.VMEM((M_PER, H), jnp.float32),
            pltpu.VMEM((M_PER, H), jnp.float32),
            pltpu.VMEM((M_PER, H), jnp.float32),
            pltpu.VMEM((M_PER, H), jnp.float32),
            pltpu.SemaphoreType.DMA((2,)),
            pltpu.SemaphoreType.DMA((2,)),
            pltpu.SemaphoreType.DMA((NS,)),
            pltpu.SemaphoreType.DMA((NS,)),
            pltpu.SemaphoreType.DMA((NS,)),
            pltpu.SemaphoreType.DMA((NS,)),
            pltpu.SemaphoreType.DMA,
            pltpu.SemaphoreType.DMA,
            pltpu.SemaphoreType.REGULAR,
            pltpu.SemaphoreType.REGULAR,
        ],
        compiler_params=pltpu.CompilerParams(
            collective_id=0,
            vmem_limit_bytes=100 * 1024 * 1024,
        ),
    )(x, w_mat)


# device time: 575811 ns/iter; 2.1704x vs baseline; 1.0046x over previous
import jax
import jax.numpy as jnp
from jax import lax
from jax.experimental import pallas as pl
from jax.experimental.pallas import tpu as pltpu

N_DEV = 4
M_PER = 1024
K = 1024
N_COLS = 8192
W = 1024
H = W // 2
S = N_COLS // W
NS = 2
RS = M_PER // NS
N_GEMMS = 4 * S
CHUNK_ORDER = (3, 1, 2, 0)
_MESH = pl.DeviceIdType.MESH


def kernel(x, w_mat):
    def body(x_hbm, w_hbm, out_ref,
             xst, wst, pbuf, send_r, recv_r, send_l, recv_l,
             xsem, wsem,
             send_sem_r, recv_sem_r, send_sem_l, recv_sem_l,
             out_sem_r, out_sem_l, credit_r, credit_l):
        q = lax.axis_index("i")
        left = (q + N_DEV - 1) % N_DEV
        right = (q + 1) % N_DEV

        barrier = pltpu.get_barrier_semaphore()
        for nbr in (left, right):
            pl.semaphore_signal(barrier, inc=1, device_id=(nbr,),
                                device_id_type=_MESH)
        pl.semaphore_wait(barrier, 2)
        pl.semaphore_signal(credit_r, inc=NS, device_id=(left,),
                            device_id_type=_MESH)
        pl.semaphore_signal(credit_l, inc=NS, device_id=(right,),
                            device_id_type=_MESH)

        pending_x = {}
        pending_w = {}
        pending_out = {}
        rd_r = {}
        rd_l = {}

        def chunk_of(g):
            return (q + CHUNK_ORDER[g % 4]) % N_DEV

        def start_xfetch(g):
            cp = pltpu.make_async_copy(
                x_hbm.at[pl.ds(chunk_of(g) * M_PER, M_PER), :],
                xst.at[g % 2], xsem.at[g % 2])
            cp.start()
            pending_x[g] = cp

        def start_wfetch(j):
            cp = pltpu.make_async_copy(
                w_hbm.at[:, pl.ds(j * W, W)], wst.at[j % 2], wsem.at[j % 2])
            cp.start()
            pending_w[j] = cp

        def emit_gemm(g):
            j = g // 4
            if g + 1 < N_GEMMS:
                start_xfetch(g + 1)
            if j in pending_w:
                pending_w.pop(j).wait()
            pending_x.pop(g).wait()
            pbuf[j % 2, pl.ds(chunk_of(g) * M_PER, M_PER), :] = jnp.dot(
                xst[g % 2, :, :], wst[j % 2, :, :],
                preferred_element_type=jnp.float32)

        def start_fwd(j, h, s):
            slot = j % 2
            if h == 0:
                src_r = pbuf.at[slot,
                                pl.ds(((q + 3) % N_DEV) * M_PER + s * RS, RS),
                                pl.ds(H, H)]
                src_l = pbuf.at[slot,
                                pl.ds(((q + 1) % N_DEV) * M_PER + s * RS, RS),
                                pl.ds(0, H)]
            else:
                src_r = send_r.at[pl.ds(s * RS, RS), :]
                src_l = send_l.at[pl.ds(s * RS, RS), :]
            pl.semaphore_wait(credit_r, 1)
            d = pltpu.make_async_remote_copy(
                src_ref=src_r, dst_ref=recv_r.at[pl.ds(s * RS, RS), :],
                send_sem=send_sem_r.at[s], recv_sem=recv_sem_r.at[s],
                device_id=(right,), device_id_type=_MESH)
            d.start()
            rd_r[(j, h, s)] = d
            pl.semaphore_wait(credit_l, 1)
            d = pltpu.make_async_remote_copy(
                src_ref=src_l, dst_ref=recv_l.at[pl.ds(s * RS, RS), :],
                send_sem=send_sem_l.at[s], recv_sem=recv_sem_l.at[s],
                device_id=(left,), device_id_type=_MESH)
            d.start()
            rd_l[(j, h, s)] = d

        def consume(j, h, s):
            slot = j % 2
            dr = rd_r.pop((j, h, s))
            dl = rd_l.pop((j, h, s))
            dr.wait_recv()
            dr.wait_send()
            dl.wait_recv()
            dl.wait_send()
            if h == 0 and s == 0:
                for d in pending_out.values():
                    d.wait()
                pending_out.clear()
            rows = pl.ds(s * RS, RS)
            cr = (q + 2 - h) % N_DEV
            cl = (q + 2 + h) % N_DEV
            loc_r = pbuf[slot, pl.ds(cr * M_PER + s * RS, RS), pl.ds(H, H)]
            loc_l = pbuf[slot, pl.ds(cl * M_PER + s * RS, RS), pl.ds(0, H)]
            if h < N_HOPS_LAST:
                send_r[rows, :] = recv_r[rows, :] + loc_r
                send_l[rows, :] = recv_l[rows, :] + loc_l
            else:
                send_r[rows, :] = jnp.maximum(recv_r[rows, :] + loc_r, 0.0)
                send_l[rows, :] = jnp.maximum(recv_l[rows, :] + loc_l, 0.0)
            if not (j == S - 1 and h == N_HOPS_LAST):
                pl.semaphore_signal(credit_r, inc=1, device_id=(left,),
                                    device_id_type=_MESH)
                pl.semaphore_signal(credit_l, inc=1, device_id=(right,),
                                    device_id_type=_MESH)
            nj, nh = (j, h + 1) if h < N_HOPS_LAST else (j + 1, 0)
            if nj < S:
                start_fwd(nj, nh, s)
            if j + 1 < S:
                base = 4 * (j + 1)
                for off in {(0, 0): (0,), (0, 1): (1,),
                            (1, 0): (2,), (2, 0): (3,)}.get((h, s), ()):
                    emit_gemm(base + off)
            if h == N_HOPS_LAST and s == NS - 1:
                d = pltpu.make_async_copy(
                    send_r, out_ref.at[:, pl.ds(j * W + H, H)], out_sem_r)
                d.start()
                pending_out['r'] = d
                d = pltpu.make_async_copy(
                    send_l, out_ref.at[:, pl.ds(j * W, H)], out_sem_l)
                d.start()
                pending_out['l'] = d

        N_HOPS_LAST = N_DEV - 2
        start_wfetch(0)
        c_r0 = (q + 3) % N_DEV
        c_l0 = (q + 1) % N_DEV
        f_a = pltpu.make_async_copy(
            x_hbm.at[pl.ds(c_r0 * M_PER, M_PER), :], xst.at[0], xsem.at[0])
        f_a.start()
        f_b = pltpu.make_async_copy(
            x_hbm.at[pl.ds(c_l0 * M_PER, M_PER), :], xst.at[1], xsem.at[1])
        f_b.start()
        pending_w.pop(0).wait()
        f_a.wait()
        f_b.wait()
        for s in range(NS):
            pbuf[0, pl.ds(c_r0 * M_PER + s * RS, RS), pl.ds(H, H)] = jnp.dot(
                xst[0, s * RS:(s + 1) * RS, :], wst[0, :, H:],
                preferred_element_type=jnp.float32)
            pbuf[0, pl.ds(c_l0 * M_PER + s * RS, RS), pl.ds(0, H)] = jnp.dot(
                xst[1, s * RS:(s + 1) * RS, :], wst[0, :, :H],
                preferred_element_type=jnp.float32)
            start_fwd(0, 0, s)
        pbuf[0, pl.ds(c_r0 * M_PER, M_PER), pl.ds(0, H)] = jnp.dot(
            xst[0, :, :], wst[0, :, :H], preferred_element_type=jnp.float32)
        pbuf[0, pl.ds(c_l0 * M_PER, M_PER), pl.ds(H, H)] = jnp.dot(
            xst[1, :, :], wst[0, :, H:], preferred_element_type=jnp.float32)
        f_c = pltpu.make_async_copy(
            x_hbm.at[pl.ds(((q + 2) % N_DEV) * M_PER, M_PER), :],
            xst.at[0], xsem.at[0])
        f_c.start()
        f_d = pltpu.make_async_copy(
            x_hbm.at[pl.ds(q * M_PER, M_PER), :], xst.at[1], xsem.at[1])
        f_d.start()
        f_c.wait()
        pbuf[0, pl.ds(((q + 2) % N_DEV) * M_PER, M_PER), :] = jnp.dot(
            xst[0, :, :], wst[0, :, :], preferred_element_type=jnp.float32)
        start_xfetch(4)
        f_d.wait()
        pbuf[0, pl.ds(q * M_PER, M_PER), :] = jnp.dot(
            xst[1, :, :], wst[0, :, :], preferred_element_type=jnp.float32)
        for j in range(S):
            if j + 1 < S:
                start_wfetch(j + 1)
            for h in range(N_DEV - 1):
                for s in range(NS):
                    consume(j, h, s)
        for d in pending_out.values():
            d.wait()

    return pl.pallas_call(
        body,
        out_shape=jax.ShapeDtypeStruct((M_PER, N_COLS), jnp.float32),
        in_specs=[pl.BlockSpec(memory_space=pl.ANY),
                  pl.BlockSpec(memory_space=pl.ANY)],
        out_specs=pl.BlockSpec(memory_space=pl.ANY),
        scratch_shapes=[
            pltpu.VMEM((2, M_PER, K), jnp.float32),
            pltpu.VMEM((2, K, W), jnp.float32),
            pltpu.VMEM((2, N_DEV * M_PER, W), jnp.float32),
            pltpu.VMEM((M_PER, H), jnp.float32),
            pltpu.VMEM((M_PER, H), jnp.float32),
            pltpu.VMEM((M_PER, H), jnp.float32),
            pltpu.VMEM((M_PER, H), jnp.float32),
            pltpu.SemaphoreType.DMA((2,)),
            pltpu.SemaphoreType.DMA((2,)),
            pltpu.SemaphoreType.DMA((NS,)),
            pltpu.SemaphoreType.DMA((NS,)),
            pltpu.SemaphoreType.DMA((NS,)),
            pltpu.SemaphoreType.DMA((NS,)),
            pltpu.SemaphoreType.DMA,
            pltpu.SemaphoreType.DMA,
            pltpu.SemaphoreType.REGULAR,
            pltpu.SemaphoreType.REGULAR,
        ],
        compiler_params=pltpu.CompilerParams(
            collective_id=0,
            vmem_limit_bytes=100 * 1024 * 1024,
        ),
    )(x, w_mat)


# device time: 575703 ns/iter; 2.1708x vs baseline; 1.0002x over previous
import jax
import jax.numpy as jnp
from jax import lax
from jax.experimental import pallas as pl
from jax.experimental.pallas import tpu as pltpu

N_DEV = 4
M_PER = 1024
K = 1024
N_COLS = 8192
W = 1024
H = W // 2
S = N_COLS // W
NS = 4
RS = M_PER // NS
N_GEMMS = 4 * S
CHUNK_ORDER = (3, 1, 2, 0)
_MESH = pl.DeviceIdType.MESH


def kernel(x, w_mat):
    def body(x_hbm, w_hbm, out_ref,
             xst, wst, pbuf, send_r, recv_r, send_l, recv_l,
             xsem, wsem,
             send_sem_r, recv_sem_r, send_sem_l, recv_sem_l,
             out_sem_r, out_sem_l, credit_r, credit_l):
        q = lax.axis_index("i")
        left = (q + N_DEV - 1) % N_DEV
        right = (q + 1) % N_DEV

        barrier = pltpu.get_barrier_semaphore()
        for nbr in (left, right):
            pl.semaphore_signal(barrier, inc=1, device_id=(nbr,),
                                device_id_type=_MESH)
        pl.semaphore_wait(barrier, 2)
        pl.semaphore_signal(credit_r, inc=NS, device_id=(left,),
                            device_id_type=_MESH)
        pl.semaphore_signal(credit_l, inc=NS, device_id=(right,),
                            device_id_type=_MESH)

        pending_x = {}
        pending_w = {}
        pending_out = {}
        rd_r = {}
        rd_l = {}

        def chunk_of(g):
            return (q + CHUNK_ORDER[g % 4]) % N_DEV

        def start_xfetch(g):
            cp = pltpu.make_async_copy(
                x_hbm.at[pl.ds(chunk_of(g) * M_PER, M_PER), :],
                xst.at[g % 2], xsem.at[g % 2])
            cp.start()
            pending_x[g] = cp

        def start_wfetch(j):
            cp = pltpu.make_async_copy(
                w_hbm.at[:, pl.ds(j * W, W)], wst.at[j % 2], wsem.at[j % 2])
            cp.start()
            pending_w[j] = cp

        def emit_gemm(g):
            j = g // 4
            if g + 1 < N_GEMMS:
                start_xfetch(g + 1)
            if j in pending_w:
                pending_w.pop(j).wait()
            pending_x.pop(g).wait()
            pbuf[j % 2, pl.ds(chunk_of(g) * M_PER, M_PER), :] = jnp.dot(
                xst[g % 2, :, :], wst[j % 2, :, :],
                preferred_element_type=jnp.float32)

        def start_fwd(j, h, s):
            slot = j % 2
            if h == 0:
                src_r = pbuf.at[slot,
                                pl.ds(((q + 3) % N_DEV) * M_PER + s * RS, RS),
                                pl.ds(H, H)]
                src_l = pbuf.at[slot,
                                pl.ds(((q + 1) % N_DEV) * M_PER + s * RS, RS),
                                pl.ds(0, H)]
            else:
                src_r = send_r.at[pl.ds(s * RS, RS), :]
                src_l = send_l.at[pl.ds(s * RS, RS), :]
            pl.semaphore_wait(credit_r, 1)
            d = pltpu.make_async_remote_copy(
                src_ref=src_r, dst_ref=recv_r.at[pl.ds(s * RS, RS), :],
                send_sem=send_sem_r.at[s], recv_sem=recv_sem_r.at[s],
                device_id=(right,), device_id_type=_MESH)
            d.start()
            rd_r[(j, h, s)] = d
            pl.semaphore_wait(credit_l, 1)
            d = pltpu.make_async_remote_copy(
                src_ref=src_l, dst_ref=recv_l.at[pl.ds(s * RS, RS), :],
                send_sem=send_sem_l.at[s], recv_sem=recv_sem_l.at[s],
                device_id=(left,), device_id_type=_MESH)
            d.start()
            rd_l[(j, h, s)] = d

        def consume(j, h, s):
            slot = j % 2
            dr = rd_r.pop((j, h, s))
            dl = rd_l.pop((j, h, s))
            dr.wait_recv()
            dr.wait_send()
            dl.wait_recv()
            dl.wait_send()
            if h == 0 and s == 0:
                for d in pending_out.values():
                    d.wait()
                pending_out.clear()
            rows = pl.ds(s * RS, RS)
            cr = (q + 2 - h) % N_DEV
            cl = (q + 2 + h) % N_DEV
            loc_r = pbuf[slot, pl.ds(cr * M_PER + s * RS, RS), pl.ds(H, H)]
            loc_l = pbuf[slot, pl.ds(cl * M_PER + s * RS, RS), pl.ds(0, H)]
            if h < N_HOPS_LAST:
                send_r[rows, :] = recv_r[rows, :] + loc_r
                send_l[rows, :] = recv_l[rows, :] + loc_l
            else:
                send_r[rows, :] = jnp.maximum(recv_r[rows, :] + loc_r, 0.0)
                send_l[rows, :] = jnp.maximum(recv_l[rows, :] + loc_l, 0.0)
            if not (j == S - 1 and h == N_HOPS_LAST):
                pl.semaphore_signal(credit_r, inc=1, device_id=(left,),
                                    device_id_type=_MESH)
                pl.semaphore_signal(credit_l, inc=1, device_id=(right,),
                                    device_id_type=_MESH)
            nj, nh = (j, h + 1) if h < N_HOPS_LAST else (j + 1, 0)
            if nj < S:
                start_fwd(nj, nh, s)
            if j + 1 < S:
                base = 4 * (j + 1)
                for off in {(0, 0): (0,), (0, 1): (1,),
                            (1, 0): (2,), (2, 0): (3,)}.get((h, s), ()):
                    emit_gemm(base + off)
            if h == N_HOPS_LAST and s == NS - 1:
                d = pltpu.make_async_copy(
                    send_r, out_ref.at[:, pl.ds(j * W + H, H)], out_sem_r)
                d.start()
                pending_out['r'] = d
                d = pltpu.make_async_copy(
                    send_l, out_ref.at[:, pl.ds(j * W, H)], out_sem_l)
                d.start()
                pending_out['l'] = d

        N_HOPS_LAST = N_DEV - 2
        start_wfetch(0)
        c_r0 = (q + 3) % N_DEV
        c_l0 = (q + 1) % N_DEV
        f_a = pltpu.make_async_copy(
            x_hbm.at[pl.ds(c_r0 * M_PER, M_PER), :], xst.at[0], xsem.at[0])
        f_a.start()
        f_b = pltpu.make_async_copy(
            x_hbm.at[pl.ds(c_l0 * M_PER, M_PER), :], xst.at[1], xsem.at[1])
        f_b.start()
        pending_w.pop(0).wait()
        f_a.wait()
        f_b.wait()
        for s in range(NS):
            pbuf[0, pl.ds(c_r0 * M_PER + s * RS, RS), pl.ds(H, H)] = jnp.dot(
                xst[0, s * RS:(s + 1) * RS, :], wst[0, :, H:],
                preferred_element_type=jnp.float32)
            pbuf[0, pl.ds(c_l0 * M_PER + s * RS, RS), pl.ds(0, H)] = jnp.dot(
                xst[1, s * RS:(s + 1) * RS, :], wst[0, :, :H],
                preferred_element_type=jnp.float32)
            start_fwd(0, 0, s)
        pbuf[0, pl.ds(c_r0 * M_PER, M_PER), pl.ds(0, H)] = jnp.dot(
            xst[0, :, :], wst[0, :, :H], preferred_element_type=jnp.float32)
        pbuf[0, pl.ds(c_l0 * M_PER, M_PER), pl.ds(H, H)] = jnp.dot(
            xst[1, :, :], wst[0, :, H:], preferred_element_type=jnp.float32)
        f_c = pltpu.make_async_copy(
            x_hbm.at[pl.ds(((q + 2) % N_DEV) * M_PER, M_PER), :],
            xst.at[0], xsem.at[0])
        f_c.start()
        f_d = pltpu.make_async_copy(
            x_hbm.at[pl.ds(q * M_PER, M_PER), :], xst.at[1], xsem.at[1])
        f_d.start()
        f_c.wait()
        pbuf[0, pl.ds(((q + 2) % N_DEV) * M_PER, M_PER), :] = jnp.dot(
            xst[0, :, :], wst[0, :, :], preferred_element_type=jnp.float32)
        start_xfetch(4)
        f_d.wait()
        pbuf[0, pl.ds(q * M_PER, M_PER), :] = jnp.dot(
            xst[1, :, :], wst[0, :, :], preferred_element_type=jnp.float32)
        for j in range(S):
            if j + 1 < S:
                start_wfetch(j + 1)
            for h in range(N_DEV - 1):
                for s in range(NS):
                    consume(j, h, s)
        for d in pending_out.values():
            d.wait()

    return pl.pallas_call(
        body,
        out_shape=jax.ShapeDtypeStruct((M_PER, N_COLS), jnp.float32),
        in_specs=[pl.BlockSpec(memory_space=pl.ANY),
                  pl.BlockSpec(memory_space=pl.ANY)],
        out_specs=pl.BlockSpec(memory_space=pl.ANY),
        scratch_shapes=[
            pltpu.VMEM((2, M_PER, K), jnp.float32),
            pltpu.VMEM((2, K, W), jnp.float32),
            pltpu.VMEM((2, N_DEV * M_PER, W), jnp.float32),
            pltpu.VMEM((M_PER, H), jnp.float32),
            pltpu.VMEM((M_PER, H), jnp.float32),
            pltpu.VMEM((M_PER, H), jnp.float32),
            pltpu.VMEM((M_PER, H), jnp.float32),
            pltpu.SemaphoreType.DMA((2,)),
            pltpu.SemaphoreType.DMA((2,)),
            pltpu.SemaphoreType.DMA((NS,)),
            pltpu.SemaphoreType.DMA((NS,)),
            pltpu.SemaphoreType.DMA((NS,)),
            pltpu.SemaphoreType.DMA((NS,)),
            pltpu.SemaphoreType.DMA,
            pltpu.SemaphoreType.DMA,
            pltpu.SemaphoreType.REGULAR,
            pltpu.SemaphoreType.REGULAR,
        ],
        compiler_params=pltpu.CompilerParams(
            collective_id=0,
            vmem_limit_bytes=100 * 1024 * 1024,
        ),
    )(x, w_mat)
